# baseline (speedup 1.0000x reference)
import functools

import jax
import jax.numpy as jnp
from jax import lax
from jax.experimental import pallas as pl
from jax.experimental.pallas import tpu as pltpu
from jax.experimental.pallas import tpu_sc as plsc

D_MODEL = 2048
K = 32
EPS = 1e-5
ROWS_PER_BLOCK = 256
NC = 2
NS = 16
NW = NC * NS
ROWS = 2 * 4096
HALF = ROWS // 2
RPW = HALF // NW
CH = 16
NV = D_MODEL // 16
NGV = NV // 16


def _norm_bits_body(x_ref, w_ref, bits_ref, gmax_ref):
    xm = x_ref[...]
    w = w_ref[...]
    mean = jnp.mean(xm, axis=1, keepdims=True)
    xc = xm - mean
    var = jnp.mean(xc * xc, axis=1, keepdims=True)
    xn = xc * lax.rsqrt(var + EPS) * w
    bits = lax.bitcast_convert_type(xn, jnp.int32) & jnp.int32(0x7FFFFFFF)
    bits_ref[...] = bits
    gmax_ref[...] = jnp.max(bits.reshape(bits.shape[0], 16, NV), axis=1)


def _finalize_body(x_ref, w_ref, g_ref, t_ref, o_ref):
    xm = x_ref[...]
    w = w_ref[...]
    g = g_ref[...]
    t = t_ref[...]
    mean = jnp.mean(xm, axis=1, keepdims=True)
    xc = xm - mean
    var = jnp.mean(xc * xc, axis=1, keepdims=True)
    xn = xc * lax.rsqrt(var + EPS) * w
    bits = lax.bitcast_convert_type(xn, jnp.int32) & jnp.int32(0x7FFFFFFF)
    keep = bits >= t
    o_ref[...] = xm + jnp.where(keep, xn * g, 0.0)


def _merge16(a, b):
    rb = lax.rev(b, (0,))
    lo = jnp.minimum(a, rb)
    hi = jnp.maximum(a, rb)
    return jnp.sort(lo), jnp.sort(hi)


def _merge32_top32(A, B):
    a1, a2 = A
    b1, b2 = B
    h1 = jnp.maximum(a1, lax.rev(b2, (0,)))
    h2 = jnp.maximum(a2, lax.rev(b1, (0,)))
    lo = jnp.minimum(h1, h2)
    hi = jnp.maximum(h1, h2)
    return jnp.sort(lo), jnp.sort(hi)


def _tournament_top32(sorted_vregs):
    m = [_merge16(sorted_vregs[2 * i], sorted_vregs[2 * i + 1])
         for i in range(len(sorted_vregs) // 2)]
    while len(m) > 1:
        m = [_merge32_top32(m[2 * i], m[2 * i + 1]) for i in range(len(m) // 2)]
    return m[0]


def _sc_select_body(bits_hbm, gmax_hbm, thr_hbm, buf0, buf1, gb0, gb1,
                    gidx, thr_loc, sem0, sem1, gsem0, gsem1):
    wid = lax.axis_index("s") * NC + lax.axis_index("c")
    base = wid * RPW
    lane = lax.iota(jnp.int32, 16)

    def copy_in(ci, buf, sem):
        return pltpu.make_async_copy(
            bits_hbm.at[pl.ds(base + ci * CH, CH)], buf, sem)

    def gcopy_in(ci, gb, gsem):
        return pltpu.make_async_copy(
            gmax_hbm.at[pl.ds(base + ci * CH, CH)], gb, gsem)

    copy_in(0, buf0, sem0).start()
    gcopy_in(0, gb0, gsem0).start()
    copy_in(1, buf1, sem1).start()
    gcopy_in(1, gb1, gsem1).start()

    def select_row(buf, gb, r):
        gv = [gb[r, pl.ds(c * 16, 16)] for c in range(NGV)]

        lo, _hi = _tournament_top32([jnp.sort(v) for v in gv])
        gt = jnp.sum(jnp.where(lane == 0, lo, 0))

        offs = jnp.full((16,), -1, jnp.int32)
        offt = jnp.full((16,), -1, jnp.int32)
        stores = []
        for u in range(NGV):
            v = gv[u]
            ms = v > gt
            mt = v == gt
            poss = offs + plsc.cumsum(jnp.where(ms, 1, 0))
            post = offt + plsc.cumsum(jnp.where(mt, 1, 0))
            stores.append((u, ms, mt, poss, post))
            offs = offs + plsc.all_reduce_population_count(ms)
            offt = offt + plsc.all_reduce_population_count(mt)
        for u, ms, mt, poss, post in stores:
            plsc.store_scatter(gidx, [31 - post], u * 16 + lane,
                               mask=mt & (post < 32))
        for u, ms, mt, poss, post in stores:
            plsc.store_scatter(gidx, [poss], u * 16 + lane, mask=ms)

        g0 = gidx[pl.ds(0, 16)]
        g1 = gidx[pl.ds(16, 16)]
        rsplat = jnp.full((16,), r, jnp.int32)
        svs = []
        for j in range(32):
            gsrc = g0 if j < 16 else g1
            gid = jnp.sum(jnp.where(lane == (j % 16), gsrc, 0))
            col = plsc.load_gather(buf, [rsplat, gid + 128 * lane])
            svs.append(jnp.sort(col))
        lo2, _hi2 = _tournament_top32(svs)
        return jnp.sum(jnp.where(lane == 0, lo2, 0))

    def process(buf, gb, ci):
        def row_body(r, _):
            t = select_row(buf, gb, r)
            plsc.store_scatter(
                thr_loc, [jnp.full((16,), ci * CH + r, jnp.int32)],
                jnp.full((16,), t, jnp.int32), mask=lane == 0)
            return 0

        lax.fori_loop(0, CH, row_body, 0)

    NCHUNK = RPW // CH

    def chunk_pair_body(i, _):
        ci0 = 2 * i
        copy_in(ci0, buf0, sem0).wait()
        gcopy_in(ci0, gb0, gsem0).wait()
        process(buf0, gb0, ci0)

        @pl.when(i < NCHUNK // 2 - 1)
        def _():
            copy_in(ci0 + 2, buf0, sem0).start()
            gcopy_in(ci0 + 2, gb0, gsem0).start()

        copy_in(ci0 + 1, buf1, sem1).wait()
        gcopy_in(ci0 + 1, gb1, gsem1).wait()
        process(buf1, gb1, ci0 + 1)

        @pl.when(i < NCHUNK // 2 - 1)
        def _():
            copy_in(ci0 + 3, buf1, sem1).start()
            gcopy_in(ci0 + 3, gb1, gsem1).start()

        return 0

    lax.fori_loop(0, NCHUNK // 2, chunk_pair_body, 0)
    pltpu.sync_copy(thr_loc, thr_hbm.at[pl.ds(base, RPW)])


_sc_select = functools.partial(
    pl.kernel,
    out_type=jax.ShapeDtypeStruct((HALF,), jnp.int32),
    mesh=plsc.VectorSubcoreMesh(core_axis_name="c", subcore_axis_name="s"),
    scratch_types=[
        pltpu.VMEM((CH, D_MODEL), jnp.int32),
        pltpu.VMEM((CH, D_MODEL), jnp.int32),
        pltpu.VMEM((CH, NV), jnp.int32),
        pltpu.VMEM((CH, NV), jnp.int32),
        pltpu.VMEM((32,), jnp.int32),
        pltpu.VMEM((RPW,), jnp.int32),
        pltpu.SemaphoreType.DMA,
        pltpu.SemaphoreType.DMA,
        pltpu.SemaphoreType.DMA,
        pltpu.SemaphoreType.DMA,
    ],
    compiler_params=pltpu.CompilerParams(needs_layout_passes=False),
)(_sc_select_body)


@jax.jit
def kernel(x, norm_weight, gamma):
    B, S, D = x.shape
    rows = B * S
    x2 = x.reshape(rows, D)
    w2 = norm_weight.reshape(1, D)
    g2 = gamma.reshape(1, D)
    grid_h = (HALF // ROWS_PER_BLOCK,)

    def norm_half(xh):
        return pl.pallas_call(
            _norm_bits_body,
            grid=grid_h,
            in_specs=[
                pl.BlockSpec((ROWS_PER_BLOCK, D), lambda i: (i, 0)),
                pl.BlockSpec((1, D), lambda i: (0, 0)),
            ],
            out_specs=[
                pl.BlockSpec((ROWS_PER_BLOCK, D), lambda i: (i, 0)),
                pl.BlockSpec((ROWS_PER_BLOCK, NV), lambda i: (i, 0)),
            ],
            out_shape=[
                jax.ShapeDtypeStruct((HALF, D), jnp.int32),
                jax.ShapeDtypeStruct((HALF, NV), jnp.int32),
            ],
            compiler_params=pltpu.CompilerParams(
                dimension_semantics=("arbitrary",),
            ),
        )(xh, w2)

    thrs = []
    for h in range(2):
        bits_h, gmax_h = norm_half(
            lax.slice(x2, (h * HALF, 0), ((h + 1) * HALF, D)))
        thrs.append(_sc_select(bits_h, gmax_h))
    thr = jnp.concatenate(thrs)
    grid = (rows // ROWS_PER_BLOCK,)

    out = pl.pallas_call(
        _finalize_body,
        grid=grid,
        in_specs=[
            pl.BlockSpec((ROWS_PER_BLOCK, D), lambda i: (i, 0)),
            pl.BlockSpec((1, D), lambda i: (0, 0)),
            pl.BlockSpec((1, D), lambda i: (0, 0)),
            pl.BlockSpec((ROWS_PER_BLOCK, 1), lambda i: (i, 0)),
        ],
        out_specs=pl.BlockSpec((ROWS_PER_BLOCK, D), lambda i: (i, 0)),
        out_shape=jax.ShapeDtypeStruct((rows, D), x.dtype),
        compiler_params=pltpu.CompilerParams(
            dimension_semantics=("arbitrary",),
        ),
    )(x2, w2, g2, thr.reshape(rows, 1))
    return out.reshape(B, S, D)


# speedup vs baseline: 1.0395x; 1.0395x over previous
import functools

import jax
import jax.numpy as jnp
from jax import lax
from jax.experimental import pallas as pl
from jax.experimental.pallas import tpu as pltpu
from jax.experimental.pallas import tpu_sc as plsc

D_MODEL = 2048
K = 32
EPS = 1e-5
ROWS_PER_BLOCK = 256
NC = 2
NS = 16
NW = NC * NS
ROWS = 2 * 4096
RPW = ROWS // NW
CH = 16
NV = D_MODEL // 16
NGV = NV // 16


def _norm_bits_body(x_ref, w_ref, bits_ref, gmax_ref):
    xm = x_ref[...]
    w = w_ref[...]
    mean = jnp.mean(xm, axis=1, keepdims=True)
    xc = xm - mean
    var = jnp.mean(xc * xc, axis=1, keepdims=True)
    xn = xc * lax.rsqrt(var + EPS) * w
    bits = lax.bitcast_convert_type(xn, jnp.int32) & jnp.int32(0x7FFFFFFF)
    bits_ref[...] = bits
    gmax_ref[...] = jnp.max(bits.reshape(bits.shape[0], 16, NV), axis=1)


def _finalize_body(x_ref, w_ref, g_ref, t_ref, o_ref):
    xm = x_ref[...]
    w = w_ref[...]
    g = g_ref[...]
    t = t_ref[...]
    mean = jnp.mean(xm, axis=1, keepdims=True)
    xc = xm - mean
    var = jnp.mean(xc * xc, axis=1, keepdims=True)
    xn = xc * lax.rsqrt(var + EPS) * w
    bits = lax.bitcast_convert_type(xn, jnp.int32) & jnp.int32(0x7FFFFFFF)
    keep = bits >= t
    o_ref[...] = xm + jnp.where(keep, xn * g, 0.0)


def _merge16(a, b):
    rb = lax.rev(b, (0,))
    lo = jnp.minimum(a, rb)
    hi = jnp.maximum(a, rb)
    return jnp.sort(lo), jnp.sort(hi)


def _merge32_top32(A, B):
    a1, a2 = A
    b1, b2 = B
    h1 = jnp.maximum(a1, lax.rev(b2, (0,)))
    h2 = jnp.maximum(a2, lax.rev(b1, (0,)))
    lo = jnp.minimum(h1, h2)
    hi = jnp.maximum(h1, h2)
    return jnp.sort(lo), jnp.sort(hi)


def _tournament_top32(sorted_vregs):
    m = [_merge16(sorted_vregs[2 * i], sorted_vregs[2 * i + 1])
         for i in range(len(sorted_vregs) // 2)]
    while len(m) > 1:
        m = [_merge32_top32(m[2 * i], m[2 * i + 1]) for i in range(len(m) // 2)]
    return m[0]


def _sc_select_body(bits_hbm, gmax_hbm, thr_hbm, buf0, buf1, gb0, gb1,
                    gidx, thr_loc, sem0, sem1, gsem0, gsem1):
    wid = lax.axis_index("s") * NC + lax.axis_index("c")
    base = wid * RPW
    lane = lax.iota(jnp.int32, 16)

    def copy_in(ci, buf, sem):
        return pltpu.make_async_copy(
            bits_hbm.at[pl.ds(base + ci * CH, CH)], buf, sem)

    def gcopy_in(ci, gb, gsem):
        return pltpu.make_async_copy(
            gmax_hbm.at[pl.ds(base + ci * CH, CH)], gb, gsem)

    copy_in(0, buf0, sem0).start()
    gcopy_in(0, gb0, gsem0).start()
    copy_in(1, buf1, sem1).start()
    gcopy_in(1, gb1, gsem1).start()

    def select_row(buf, gb, r):
        gv = [gb[r, pl.ds(c * 16, 16)] for c in range(NGV)]

        lo, _hi = _tournament_top32([jnp.sort(v) for v in gv])
        gt = jnp.sum(jnp.where(lane == 0, lo, 0))

        offs = jnp.full((16,), -1, jnp.int32)
        offt = jnp.full((16,), -1, jnp.int32)
        stores = []
        for u in range(NGV):
            v = gv[u]
            ms = v > gt
            mt = v == gt
            poss = offs + plsc.cumsum(jnp.where(ms, 1, 0))
            post = offt + plsc.cumsum(jnp.where(mt, 1, 0))
            stores.append((u, ms, mt, poss, post))
            offs = offs + plsc.all_reduce_population_count(ms)
            offt = offt + plsc.all_reduce_population_count(mt)
        for u, ms, mt, poss, post in stores:
            plsc.store_scatter(gidx, [31 - post], u * 16 + lane,
                               mask=mt & (post < 32))
        for u, ms, mt, poss, post in stores:
            plsc.store_scatter(gidx, [poss], u * 16 + lane, mask=ms)

        g0 = gidx[pl.ds(0, 16)]
        g1 = gidx[pl.ds(16, 16)]
        rsplat = jnp.full((16,), r, jnp.int32)
        svs = []
        for j in range(32):
            gsrc = g0 if j < 16 else g1
            gid = jnp.sum(jnp.where(lane == (j % 16), gsrc, 0))
            col = plsc.load_gather(buf, [rsplat, gid + 128 * lane])
            svs.append(jnp.sort(col))
        lo2, _hi2 = _tournament_top32(svs)
        return jnp.sum(jnp.where(lane == 0, lo2, 0))

    def process(buf, gb, ci):
        def row_body(r, _):
            t = select_row(buf, gb, r)
            plsc.store_scatter(
                thr_loc, [jnp.full((16,), ci * CH + r, jnp.int32)],
                jnp.full((16,), t, jnp.int32), mask=lane == 0)
            return 0

        lax.fori_loop(0, CH, row_body, 0)

    NCHUNK = RPW // CH

    def chunk_pair_body(i, _):
        ci0 = 2 * i
        copy_in(ci0, buf0, sem0).wait()
        gcopy_in(ci0, gb0, gsem0).wait()
        process(buf0, gb0, ci0)

        @pl.when(i < NCHUNK // 2 - 1)
        def _():
            copy_in(ci0 + 2, buf0, sem0).start()
            gcopy_in(ci0 + 2, gb0, gsem0).start()

        copy_in(ci0 + 1, buf1, sem1).wait()
        gcopy_in(ci0 + 1, gb1, gsem1).wait()
        process(buf1, gb1, ci0 + 1)

        @pl.when(i < NCHUNK // 2 - 1)
        def _():
            copy_in(ci0 + 3, buf1, sem1).start()
            gcopy_in(ci0 + 3, gb1, gsem1).start()

        return 0

    lax.fori_loop(0, NCHUNK // 2, chunk_pair_body, 0)
    pltpu.sync_copy(thr_loc, thr_hbm.at[pl.ds(base, RPW)])


_sc_select = functools.partial(
    pl.kernel,
    out_type=jax.ShapeDtypeStruct((ROWS,), jnp.int32),
    mesh=plsc.VectorSubcoreMesh(core_axis_name="c", subcore_axis_name="s"),
    scratch_types=[
        pltpu.VMEM((CH, D_MODEL), jnp.int32),
        pltpu.VMEM((CH, D_MODEL), jnp.int32),
        pltpu.VMEM((CH, NV), jnp.int32),
        pltpu.VMEM((CH, NV), jnp.int32),
        pltpu.VMEM((32,), jnp.int32),
        pltpu.VMEM((RPW,), jnp.int32),
        pltpu.SemaphoreType.DMA,
        pltpu.SemaphoreType.DMA,
        pltpu.SemaphoreType.DMA,
        pltpu.SemaphoreType.DMA,
    ],
    compiler_params=pltpu.CompilerParams(needs_layout_passes=False),
)(_sc_select_body)


@jax.jit
def kernel(x, norm_weight, gamma):
    B, S, D = x.shape
    rows = B * S
    x2 = x.reshape(rows, D)
    w2 = norm_weight.reshape(1, D)
    g2 = gamma.reshape(1, D)
    grid = (rows // ROWS_PER_BLOCK,)

    bits, gmax = pl.pallas_call(
        _norm_bits_body,
        grid=grid,
        in_specs=[
            pl.BlockSpec((ROWS_PER_BLOCK, D), lambda i: (i, 0)),
            pl.BlockSpec((1, D), lambda i: (0, 0)),
        ],
        out_specs=[
            pl.BlockSpec((ROWS_PER_BLOCK, D), lambda i: (i, 0)),
            pl.BlockSpec((ROWS_PER_BLOCK, NV), lambda i: (i, 0)),
        ],
        out_shape=[
            jax.ShapeDtypeStruct((rows, D), jnp.int32),
            jax.ShapeDtypeStruct((rows, NV), jnp.int32),
        ],
        compiler_params=pltpu.CompilerParams(
            dimension_semantics=("arbitrary",),
        ),
    )(x2, w2)

    thr = _sc_select(bits, gmax)

    out = pl.pallas_call(
        _finalize_body,
        grid=grid,
        in_specs=[
            pl.BlockSpec((ROWS_PER_BLOCK, D), lambda i: (i, 0)),
            pl.BlockSpec((1, D), lambda i: (0, 0)),
            pl.BlockSpec((1, D), lambda i: (0, 0)),
            pl.BlockSpec((ROWS_PER_BLOCK, 1), lambda i: (i, 0)),
        ],
        out_specs=pl.BlockSpec((ROWS_PER_BLOCK, D), lambda i: (i, 0)),
        out_shape=jax.ShapeDtypeStruct((rows, D), x.dtype),
        compiler_params=pltpu.CompilerParams(
            dimension_semantics=("arbitrary",),
        ),
    )(x2, w2, g2, thr.reshape(rows, 1))
    return out.reshape(B, S, D)
